# baseline (speedup 1.0000x reference)
import functools

import jax
import jax.numpy as jnp
from jax import lax
from jax.experimental import pallas as pl
from jax.experimental.pallas import tpu as pltpu
from jax.experimental.pallas import tpu_sc as plsc

VOCAB = 100000
EMB = 300
EMBP = 384
HID = 128
B = 4096
L = 50

NC = 2
NS = 16
NW = NC * NS
BAGS_PER_W = B // NW
CHUNK_BAGS = 2
ROWS = CHUNK_BAGS * L
ROWSP = 104
NCHUNKS = BAGS_PER_W // CHUNK_BAGS
NVEC = EMBP // 16
OST_CHUNKS = 8

_mesh = plsc.VectorSubcoreMesh(core_axis_name="c", subcore_axis_name="s")


def _gather(emb_hbm, idx_v, c, rows_ref, sem):
    return pltpu.make_async_copy(emb_hbm.at[idx_v.at[c]], rows_ref, sem)


def _accumulate(rows_ref, ost_v, c):
    for bag in range(CHUNK_BAGS):
        def body(r, accs, _bag=bag):
            row = _bag * L + r
            return tuple(
                accs[k] + rows_ref[row, pl.ds(16 * k, 16)] for k in range(NVEC)
            )

        zero = jnp.zeros((16,), jnp.float32)
        accs = lax.fori_loop(0, L, body, (zero,) * NVEC)
        slot = lax.rem(c, OST_CHUNKS) * CHUNK_BAGS + bag
        for k in range(NVEC):
            ost_v[slot, pl.ds(16 * k, 16)] = accs[k]


@functools.partial(
    pl.kernel,
    mesh=_mesh,
    out_type=jax.ShapeDtypeStruct((B, EMBP), jnp.float32),
    scratch_types=[
        pltpu.VMEM((NCHUNKS, ROWSP), jnp.int32),
        pltpu.VMEM((ROWSP, EMBP), jnp.float32),
        pltpu.VMEM((ROWSP, EMBP), jnp.float32),
        pltpu.VMEM((OST_CHUNKS * CHUNK_BAGS, EMBP), jnp.float32),
        pltpu.SemaphoreType.DMA,
        pltpu.SemaphoreType.DMA,
    ],
)
def _sc_pool(x_hbm, emb_hbm, out_hbm, idx_v, rows0, rows1, ost_v, sem0, sem1):
    sid = lax.axis_index("s")
    wid = sid * NC + lax.axis_index("c")
    base = wid * BAGS_PER_W
    pltpu.sync_copy(x_hbm.at[wid], idx_v)

    _gather(emb_hbm, idx_v, 0, rows0, sem0).start()
    _gather(emb_hbm, idx_v, 1, rows1, sem1).start()

    @pl.loop(0, NCHUNKS, step=2)
    def _(c):
        for step, (rows, sem) in enumerate(((rows0, sem0), (rows1, sem1))):
            cc = c + step
            _gather(emb_hbm, idx_v, cc, rows, sem).wait()
            _accumulate(rows, ost_v, cc)

            @pl.when(cc + 2 < NCHUNKS)
            def _():
                _gather(emb_hbm, idx_v, cc + 2, rows, sem).start()

            @pl.when(lax.rem(cc, OST_CHUNKS) == OST_CHUNKS - 1)
            def _():
                flush = OST_CHUNKS * CHUNK_BAGS
                off = pl.multiple_of(base + (cc + 1) * CHUNK_BAGS - flush, 8)
                pltpu.sync_copy(ost_v, out_hbm.at[pl.ds(off, flush)])


_TC_BLK = 512


def _proj_body(s_ref, len_ref, wt_ref, b_ref, o_ref):
    s = s_ref[...] / len_ref[...]
    o_ref[...] = (
        jnp.dot(s, wt_ref[...], preferred_element_type=jnp.float32) + b_ref[...]
    )


def _tc_project(sums, length_f, Wt, b2):
    return pl.pallas_call(
        _proj_body,
        grid=(B // _TC_BLK,),
        in_specs=[
            pl.BlockSpec((_TC_BLK, EMBP), lambda i: (i, 0)),
            pl.BlockSpec((_TC_BLK, 1), lambda i: (i, 0)),
            pl.BlockSpec((EMBP, HID), lambda i: (0, 0)),
            pl.BlockSpec((1, HID), lambda i: (0, 0)),
        ],
        out_specs=pl.BlockSpec((_TC_BLK, HID), lambda i: (i, 0)),
        out_shape=jax.ShapeDtypeStruct((B, HID), jnp.float32),
    )(sums, length_f, Wt, b2)


@jax.jit
def kernel(x, length, emb, W, b):
    x3d = x.astype(jnp.int32).reshape(NW, NCHUNKS, ROWS)
    x3d = jnp.pad(x3d, ((0, 0), (0, 0), (0, ROWSP - ROWS)))
    embp = jnp.pad(emb, ((0, 0), (0, EMBP - EMB)))
    sums = _sc_pool(x3d, embp)
    length_f = length.astype(jnp.float32).reshape(B, 1)
    Wtp = jnp.pad(W.T, ((0, EMBP - EMB), (0, 0)))
    return _tc_project(sums, length_f, Wtp, b.reshape(1, HID))


# speedup vs baseline: 1.5099x; 1.5099x over previous
import functools

import jax
import jax.numpy as jnp
from jax import lax
from jax.experimental import pallas as pl
from jax.experimental.pallas import tpu as pltpu
from jax.experimental.pallas import tpu_sc as plsc

VOCAB = 100000
EMB = 300
EMBP = 384
HID = 128
B = 4096
L = 50

NC = 2
NS = 16
NW = NC * NS
BAGS_PER_W = B // NW
CHUNK_BAGS = 2
ROWS = CHUNK_BAGS * L
ROWSP = 104
NCHUNKS = BAGS_PER_W // CHUNK_BAGS
NVEC = EMBP // 16
OST_CHUNKS = 8

_mesh = plsc.VectorSubcoreMesh(core_axis_name="c", subcore_axis_name="s")


def _gather(emb_hbm, idx_v, c, rows_ref, sem):
    return pltpu.make_async_copy(emb_hbm.at[idx_v.at[c]], rows_ref, sem)


def _accumulate(rows_ref, ost_v, c):
    for bag in range(CHUNK_BAGS):
        def body(r, accs, _bag=bag):
            row = _bag * L + r
            return tuple(
                accs[k] + rows_ref[row, pl.ds(16 * k, 16)] for k in range(NVEC)
            )

        zero = jnp.zeros((16,), jnp.float32)
        accs = lax.fori_loop(0, L, body, (zero,) * NVEC, unroll=10)
        slot = lax.rem(c, OST_CHUNKS) * CHUNK_BAGS + bag
        for k in range(NVEC):
            ost_v[slot, pl.ds(16 * k, 16)] = accs[k]


@functools.partial(
    pl.kernel,
    mesh=_mesh,
    out_type=jax.ShapeDtypeStruct((B, EMBP), jnp.float32),
    scratch_types=[
        pltpu.VMEM((NCHUNKS, ROWSP), jnp.int32),
        pltpu.VMEM((ROWSP, EMBP), jnp.float32),
        pltpu.VMEM((ROWSP, EMBP), jnp.float32),
        pltpu.VMEM((OST_CHUNKS * CHUNK_BAGS, EMBP), jnp.float32),
        pltpu.SemaphoreType.DMA,
        pltpu.SemaphoreType.DMA,
    ],
)
def _sc_pool(x_hbm, emb_hbm, out_hbm, idx_v, rows0, rows1, ost_v, sem0, sem1):
    sid = lax.axis_index("s")
    wid = sid * NC + lax.axis_index("c")
    base = wid * BAGS_PER_W
    pltpu.sync_copy(x_hbm.at[wid], idx_v)

    _gather(emb_hbm, idx_v, 0, rows0, sem0).start()
    _gather(emb_hbm, idx_v, 1, rows1, sem1).start()

    @pl.loop(0, NCHUNKS, step=2)
    def _(c):
        for step, (rows, sem) in enumerate(((rows0, sem0), (rows1, sem1))):
            cc = c + step
            _gather(emb_hbm, idx_v, cc, rows, sem).wait()
            _accumulate(rows, ost_v, cc)

            @pl.when(cc + 2 < NCHUNKS)
            def _():
                _gather(emb_hbm, idx_v, cc + 2, rows, sem).start()

            @pl.when(lax.rem(cc, OST_CHUNKS) == OST_CHUNKS - 1)
            def _():
                flush = OST_CHUNKS * CHUNK_BAGS
                off = pl.multiple_of(base + (cc + 1) * CHUNK_BAGS - flush, 8)
                pltpu.sync_copy(ost_v, out_hbm.at[pl.ds(off, flush)])


_PAD_BLK = 2000


def _pad_body(e_ref, o_ref):
    o_ref[:, : (EMB // 128) * 128] = e_ref[:, : (EMB // 128) * 128]
    tail = jnp.pad(
        e_ref[:, (EMB // 128) * 128 :],
        ((0, 0), (0, EMBP - EMB)),
    )
    o_ref[:, (EMB // 128) * 128 :] = tail


def _tc_pad_table(emb):
    return pl.pallas_call(
        _pad_body,
        grid=(VOCAB // _PAD_BLK,),
        in_specs=[pl.BlockSpec((_PAD_BLK, EMB), lambda i: (i, 0))],
        out_specs=pl.BlockSpec((_PAD_BLK, EMBP), lambda i: (i, 0)),
        out_shape=jax.ShapeDtypeStruct((VOCAB, EMBP), jnp.float32),
    )(emb)


_TC_BLK = 512


def _proj_body(s_ref, len_ref, wt_ref, b_ref, o_ref):
    s = s_ref[...] / len_ref[...]
    o_ref[...] = (
        jnp.dot(s, wt_ref[...], preferred_element_type=jnp.float32) + b_ref[...]
    )


def _tc_project(sums, length_f, Wt, b2):
    return pl.pallas_call(
        _proj_body,
        grid=(B // _TC_BLK,),
        in_specs=[
            pl.BlockSpec((_TC_BLK, EMBP), lambda i: (i, 0)),
            pl.BlockSpec((_TC_BLK, 1), lambda i: (i, 0)),
            pl.BlockSpec((EMBP, HID), lambda i: (0, 0)),
            pl.BlockSpec((1, HID), lambda i: (0, 0)),
        ],
        out_specs=pl.BlockSpec((_TC_BLK, HID), lambda i: (i, 0)),
        out_shape=jax.ShapeDtypeStruct((B, HID), jnp.float32),
    )(sums, length_f, Wt, b2)


@jax.jit
def kernel(x, length, emb, W, b):
    x3d = x.astype(jnp.int32).reshape(NW, NCHUNKS, ROWS)
    x3d = jnp.pad(x3d, ((0, 0), (0, 0), (0, ROWSP - ROWS)))
    embp = _tc_pad_table(emb)
    sums = _sc_pool(x3d, embp)
    length_f = length.astype(jnp.float32).reshape(B, 1)
    Wtp = jnp.pad(W.T, ((0, EMBP - EMB), (0, 0)))
    return _tc_project(sums, length_f, Wtp, b.reshape(1, HID))


# speedup vs baseline: 1.5176x; 1.0051x over previous
import functools

import jax
import jax.numpy as jnp
from jax import lax
from jax.experimental import pallas as pl
from jax.experimental.pallas import tpu as pltpu
from jax.experimental.pallas import tpu_sc as plsc

VOCAB = 100000
EMB = 300
EMBP = 384
HID = 128
B = 4096
L = 50

NC = 2
NS = 16
NW = NC * NS
BAGS_PER_W = B // NW
CHUNK_BAGS = 2
ROWS = CHUNK_BAGS * L
ROWSP = 104
NCHUNKS = BAGS_PER_W // CHUNK_BAGS
NVEC = 19
OST_CHUNKS = 8

_mesh = plsc.VectorSubcoreMesh(core_axis_name="c", subcore_axis_name="s")


def _gather(emb_hbm, idx_v, c, rows_ref, sem):
    return pltpu.make_async_copy(emb_hbm.at[idx_v.at[c]], rows_ref, sem)


def _accumulate(rows_ref, ost_v, c):
    for bag in range(CHUNK_BAGS):
        def body(r, accs, _bag=bag):
            row = _bag * L + r
            return tuple(
                accs[k] + rows_ref[row, pl.ds(16 * k, 16)] for k in range(NVEC)
            )

        zero = jnp.zeros((16,), jnp.float32)
        accs = lax.fori_loop(0, L, body, (zero,) * NVEC, unroll=10)
        slot = lax.rem(c, OST_CHUNKS) * CHUNK_BAGS + bag
        for k in range(NVEC):
            ost_v[slot, pl.ds(16 * k, 16)] = accs[k]


@functools.partial(
    pl.kernel,
    mesh=_mesh,
    out_type=jax.ShapeDtypeStruct((B, EMBP), jnp.float32),
    scratch_types=[
        pltpu.VMEM((NCHUNKS, ROWSP), jnp.int32),
        pltpu.VMEM((ROWSP, EMBP), jnp.float32),
        pltpu.VMEM((ROWSP, EMBP), jnp.float32),
        pltpu.VMEM((OST_CHUNKS * CHUNK_BAGS, EMBP), jnp.float32),
        pltpu.SemaphoreType.DMA,
        pltpu.SemaphoreType.DMA,
    ],
)
def _sc_pool(x_hbm, emb_hbm, out_hbm, idx_v, rows0, rows1, ost_v, sem0, sem1):
    sid = lax.axis_index("s")
    wid = sid * NC + lax.axis_index("c")
    base = wid * BAGS_PER_W
    pltpu.sync_copy(x_hbm.at[wid], idx_v)

    _gather(emb_hbm, idx_v, 0, rows0, sem0).start()
    _gather(emb_hbm, idx_v, 1, rows1, sem1).start()

    @pl.loop(0, NCHUNKS, step=2)
    def _(c):
        for step, (rows, sem) in enumerate(((rows0, sem0), (rows1, sem1))):
            cc = c + step
            _gather(emb_hbm, idx_v, cc, rows, sem).wait()
            _accumulate(rows, ost_v, cc)

            @pl.when(cc + 2 < NCHUNKS)
            def _():
                _gather(emb_hbm, idx_v, cc + 2, rows, sem).start()

            @pl.when(lax.rem(cc, OST_CHUNKS) == OST_CHUNKS - 1)
            def _():
                flush = OST_CHUNKS * CHUNK_BAGS
                off = pl.multiple_of(base + (cc + 1) * CHUNK_BAGS - flush, 8)
                pltpu.sync_copy(ost_v, out_hbm.at[pl.ds(off, flush)])


_PAD_BLK = 2000


def _pad_body(e_ref, o_ref):
    o_ref[:, :EMB] = e_ref[...]


def _tc_pad_table(emb):
    return pl.pallas_call(
        _pad_body,
        grid=(VOCAB // _PAD_BLK,),
        in_specs=[pl.BlockSpec((_PAD_BLK, EMB), lambda i: (i, 0))],
        out_specs=pl.BlockSpec((_PAD_BLK, EMBP), lambda i: (i, 0)),
        out_shape=jax.ShapeDtypeStruct((VOCAB, EMBP), jnp.float32),
    )(emb)


_TC_BLK = 512


def _proj_body(s_ref, len_ref, wt_ref, b_ref, o_ref):
    s = s_ref[:, :EMB] / len_ref[...]
    o_ref[...] = (
        jnp.dot(s, wt_ref[:EMB, :], preferred_element_type=jnp.float32)
        + b_ref[...]
    )


def _tc_project(sums, length_f, Wt, b2):
    return pl.pallas_call(
        _proj_body,
        grid=(B // _TC_BLK,),
        in_specs=[
            pl.BlockSpec((_TC_BLK, EMBP), lambda i: (i, 0)),
            pl.BlockSpec((_TC_BLK, 1), lambda i: (i, 0)),
            pl.BlockSpec((EMBP, HID), lambda i: (0, 0)),
            pl.BlockSpec((1, HID), lambda i: (0, 0)),
        ],
        out_specs=pl.BlockSpec((_TC_BLK, HID), lambda i: (i, 0)),
        out_shape=jax.ShapeDtypeStruct((B, HID), jnp.float32),
    )(sums, length_f, Wt, b2)


@jax.jit
def kernel(x, length, emb, W, b):
    x3d = x.astype(jnp.int32).reshape(NW, NCHUNKS, ROWS)
    x3d = jnp.pad(x3d, ((0, 0), (0, 0), (0, ROWSP - ROWS)))
    embp = _tc_pad_table(emb)
    sums = _sc_pool(x3d, embp)
    length_f = length.astype(jnp.float32).reshape(B, 1)
    Wtp = jnp.pad(W.T, ((0, EMBP - EMB), (0, 0)))
    return _tc_project(sums, length_f, Wtp, b.reshape(1, HID))


# speedup vs baseline: 2.3003x; 1.5157x over previous
import functools

import jax
import jax.numpy as jnp
from jax import lax
from jax.experimental import pallas as pl
from jax.experimental.pallas import tpu as pltpu
from jax.experimental.pallas import tpu_sc as plsc

VOCAB = 100000
EMB = 300
HID = 128
B = 4096
L = 50

NC = 2
NS = 16
NW = NC * NS
BAGS_PER_W = B // NW
CHUNK_BAGS = 2
ROWS = CHUNK_BAGS * L
ROWSP = 104
NCHUNKS = BAGS_PER_W // CHUNK_BAGS
NVEC = HID // 16
OST_CHUNKS = 8

_mesh = plsc.VectorSubcoreMesh(core_axis_name="c", subcore_axis_name="s")


def _gather(tab_hbm, idx_v, c, rows_ref, sem):
    return pltpu.make_async_copy(tab_hbm.at[idx_v.at[c]], rows_ref, sem)


def _accumulate(rows_ref, ost_v, c):
    for bag in range(CHUNK_BAGS):
        def body(r, accs, _bag=bag):
            row = _bag * L + r
            return tuple(
                accs[k] + rows_ref[row, pl.ds(16 * k, 16)] for k in range(NVEC)
            )

        zero = jnp.zeros((16,), jnp.float32)
        accs = lax.fori_loop(0, L, body, (zero,) * NVEC, unroll=10)
        slot = lax.rem(c, OST_CHUNKS) * CHUNK_BAGS + bag
        for k in range(NVEC):
            ost_v[slot, pl.ds(16 * k, 16)] = accs[k]


@functools.partial(
    pl.kernel,
    mesh=_mesh,
    out_type=jax.ShapeDtypeStruct((B, HID), jnp.float32),
    scratch_types=[
        pltpu.VMEM((NCHUNKS, ROWSP), jnp.int32),
        pltpu.VMEM((ROWSP, HID), jnp.float32),
        pltpu.VMEM((ROWSP, HID), jnp.float32),
        pltpu.VMEM((OST_CHUNKS * CHUNK_BAGS, HID), jnp.float32),
        pltpu.SemaphoreType.DMA,
        pltpu.SemaphoreType.DMA,
    ],
)
def _sc_pool(x_hbm, tab_hbm, out_hbm, idx_v, rows0, rows1, ost_v, sem0, sem1):
    sid = lax.axis_index("s")
    wid = sid * NC + lax.axis_index("c")
    base = wid * BAGS_PER_W
    pltpu.sync_copy(x_hbm.at[wid], idx_v)

    _gather(tab_hbm, idx_v, 0, rows0, sem0).start()
    _gather(tab_hbm, idx_v, 1, rows1, sem1).start()

    @pl.loop(0, NCHUNKS, step=2)
    def _(c):
        for step, (rows, sem) in enumerate(((rows0, sem0), (rows1, sem1))):
            cc = c + step
            _gather(tab_hbm, idx_v, cc, rows, sem).wait()
            _accumulate(rows, ost_v, cc)

            @pl.when(cc + 2 < NCHUNKS)
            def _():
                _gather(tab_hbm, idx_v, cc + 2, rows, sem).start()

            @pl.when(lax.rem(cc, OST_CHUNKS) == OST_CHUNKS - 1)
            def _():
                flush = OST_CHUNKS * CHUNK_BAGS
                off = pl.multiple_of(base + (cc + 1) * CHUNK_BAGS - flush, 8)
                pltpu.sync_copy(ost_v, out_hbm.at[pl.ds(off, flush)])


_MM_BLK = 2048


def _mm_body(et_ref, wt_ref, o_ref):
    o_ref[...] = lax.dot_general(
        et_ref[...], wt_ref[...],
        dimension_numbers=(((0,), (0,)), ((), ())),
        preferred_element_type=jnp.float32,
    )


def _tc_project_table(embT, Wt):
    return pl.pallas_call(
        _mm_body,
        grid=(pl.cdiv(VOCAB, _MM_BLK),),
        in_specs=[
            pl.BlockSpec((EMB, _MM_BLK), lambda i: (0, i)),
            pl.BlockSpec((EMB, HID), lambda i: (0, 0)),
        ],
        out_specs=pl.BlockSpec((_MM_BLK, HID), lambda i: (i, 0)),
        out_shape=jax.ShapeDtypeStruct((VOCAB, HID), jnp.float32),
    )(embT, Wt)


def _epi_body(s_ref, len_ref, b_ref, o_ref):
    o_ref[...] = s_ref[...] / len_ref[...] + b_ref[...]


def _tc_epilogue(sums, length_f, b2):
    return pl.pallas_call(
        _epi_body,
        grid=(1,),
        in_specs=[
            pl.BlockSpec((B, HID), lambda i: (0, 0)),
            pl.BlockSpec((B, 1), lambda i: (0, 0)),
            pl.BlockSpec((1, HID), lambda i: (0, 0)),
        ],
        out_specs=pl.BlockSpec((B, HID), lambda i: (0, 0)),
        out_shape=jax.ShapeDtypeStruct((B, HID), jnp.float32),
    )(sums, length_f, b2)


@jax.jit
def kernel(x, length, emb, W, b):
    P = _tc_project_table(emb.T, W.T)
    x3d = x.astype(jnp.int32).reshape(NW, NCHUNKS, ROWS)
    x3d = jnp.pad(x3d, ((0, 0), (0, 0), (0, ROWSP - ROWS)))
    sums = _sc_pool(x3d, P)
    length_f = length.astype(jnp.float32).reshape(B, 1)
    return _tc_epilogue(sums, length_f, b.reshape(1, HID))
